# Initial kernel scaffold; baseline (speedup 1.0000x reference)
import functools

import jax
import jax.numpy as jnp
from jax import lax
from jax.experimental import pallas as pl
from jax.experimental.pallas import tpu as pltpu
from jax.experimental.pallas import tpu_sc as plsc

N_NODES = 10000
N_EDGES = 320000
D_FEAT = 128
HID = 16
N_CLS = 10
N_GRAPHS = 64

NP = 10240
PAD_NODE = N_NODES
N_TILES = 32
CHUNK = 128
EPT = 10112
E_PAD = EPT * N_TILES
NCH = EPT // CHUNK
ROWS_PER_TILE = NP // 16


@functools.lru_cache(maxsize=None)
def _sc_kernels():
    mesh = plsc.VectorSubcoreMesh(core_axis_name="c", subcore_axis_name="s")
    zeros16 = jnp.zeros((16,), jnp.float32)
    ones16 = jnp.ones((16,), jnp.float32)

    def _zero_my_slice(zbuf, acc_sh, s):
        @pl.loop(0, ROWS_PER_TILE)
        def _(i):
            zbuf[i, :] = zeros16
        pltpu.sync_copy(zbuf, acc_sh.at[pl.ds(s * ROWS_PER_TILE, ROWS_PER_TILE)])

    def _drain_my_slice(acc_sh, out_hbm, c, s):
        pltpu.sync_copy(
            acc_sh.at[pl.ds(s * ROWS_PER_TILE, ROWS_PER_TILE)],
            out_hbm.at[c].at[pl.ds(s * ROWS_PER_TILE, ROWS_PER_TILE)],
        )

    def deg_body(dst_hbm, out_hbm, didx, ones_buf, zbuf, acc_sh):
        c = lax.axis_index("c")
        s = lax.axis_index("s")

        @pl.loop(0, CHUNK)
        def _(i):
            ones_buf[i, :] = ones16

        _zero_my_slice(zbuf, acc_sh, s)
        plsc.subcore_barrier()
        base = (c * 16 + s) * EPT

        @pl.loop(0, NCH)
        def _(j):
            off = base + j * CHUNK
            pltpu.sync_copy(dst_hbm.at[pl.ds(off, CHUNK)], didx)
            pltpu.sync_copy(ones_buf, acc_sh.at[didx], add=True)

        plsc.subcore_barrier()
        _drain_my_slice(acc_sh, out_hbm, c, s)

    deg_call = pl.kernel(
        deg_body,
        out_type=jax.ShapeDtypeStruct((2, NP, HID), jnp.float32),
        mesh=mesh,
        scratch_types=[
            pltpu.VMEM((CHUNK,), jnp.int32),
            pltpu.VMEM((CHUNK, HID), jnp.float32),
            pltpu.VMEM((ROWS_PER_TILE, HID), jnp.float32),
            pltpu.VMEM_SHARED((NP, HID), jnp.float32),
        ],
    )

    def agg_body(src_hbm, dst_hbm, tab_hbm, out_hbm, sidx, didx, rows, zbuf,
                 acc_sh, gsem):
        c = lax.axis_index("c")
        s = lax.axis_index("s")
        _zero_my_slice(zbuf, acc_sh, s)
        plsc.subcore_barrier()
        base = (c * 16 + s) * EPT

        @pl.loop(0, NCH)
        def _(j):
            off = base + j * CHUNK
            pltpu.sync_copy(src_hbm.at[pl.ds(off, CHUNK)], sidx)
            pltpu.sync_copy(dst_hbm.at[pl.ds(off, CHUNK)], didx)
            pltpu.async_copy(tab_hbm.at[sidx], rows, gsem).wait()
            pltpu.sync_copy(rows, acc_sh.at[didx], add=True)

        plsc.subcore_barrier()
        _drain_my_slice(acc_sh, out_hbm, c, s)

    agg_call = pl.kernel(
        agg_body,
        out_type=jax.ShapeDtypeStruct((2, NP, HID), jnp.float32),
        mesh=mesh,
        scratch_types=[
            pltpu.VMEM((CHUNK,), jnp.int32),
            pltpu.VMEM((CHUNK,), jnp.int32),
            pltpu.VMEM((CHUNK, HID), jnp.float32),
            pltpu.VMEM((ROWS_PER_TILE, HID), jnp.float32),
            pltpu.VMEM_SHARED((NP, HID), jnp.float32),
            pltpu.SemaphoreType.DMA,
        ],
    )

    return deg_call, agg_call


def _mm1_body(x_ref, w1_ref, dega_ref, degb_ref, h1p_ref, dinv_ref):
    deg = dega_ref[...] + degb_ref[...] + 1.0
    dinv = lax.rsqrt(deg)
    h1 = jnp.dot(x_ref[...], w1_ref[...], preferred_element_type=jnp.float32)
    dinv_ref[...] = dinv
    h1p_ref[...] = h1 * dinv


def _mm1(x_pad, W1, dega, degb):
    BM = 2048
    grid = NP // BM
    return pl.pallas_call(
        _mm1_body,
        grid=(grid,),
        in_specs=[
            pl.BlockSpec((BM, D_FEAT), lambda i: (i, 0)),
            pl.BlockSpec((D_FEAT, HID), lambda i: (0, 0)),
            pl.BlockSpec((BM, HID), lambda i: (i, 0)),
            pl.BlockSpec((BM, HID), lambda i: (i, 0)),
        ],
        out_specs=[
            pl.BlockSpec((BM, HID), lambda i: (i, 0)),
            pl.BlockSpec((BM, HID), lambda i: (i, 0)),
        ],
        out_shape=[
            jax.ShapeDtypeStruct((NP, HID), jnp.float32),
            jax.ShapeDtypeStruct((NP, HID), jnp.float32),
        ],
    )(x_pad, W1, dega, degb)


def _mm2_body(agga_ref, aggb_ref, h1p_ref, dinv_ref, b1_ref, w2_ref, h2p_ref):
    dinv = dinv_ref[...]
    pre = dinv * (agga_ref[...] + aggb_ref[...] + h1p_ref[...]) + b1_ref[...]
    r = jnp.maximum(pre, 0.0)
    h2 = jnp.dot(r, w2_ref[...], preferred_element_type=jnp.float32)
    h2p_ref[...] = h2 * dinv


def _mm2(agga, aggb, h1p, dinv, b1r, W2):
    return pl.pallas_call(
        _mm2_body,
        out_shape=jax.ShapeDtypeStruct((NP, HID), jnp.float32),
    )(agga, aggb, h1p, dinv, b1r, W2)


def _final_body(agga_ref, aggb_ref, h2p_ref, dinv_ref, b2_ref, batch_ref,
                wfc_ref, bfc_ref, out_ref):
    out2 = dinv_ref[...] * (agga_ref[...] + aggb_ref[...] + h2p_ref[...]) + b2_ref[...]
    ids = lax.broadcasted_iota(jnp.int32, (N_GRAPHS, NP), 0)
    sel = (ids == batch_ref[...]).astype(jnp.float32)
    sums = jnp.dot(sel, out2, preferred_element_type=jnp.float32)
    counts = jnp.sum(sel, axis=1, keepdims=True)
    mean = sums / jnp.maximum(counts, 1.0)
    out_ref[...] = jnp.dot(mean, wfc_ref[...],
                           preferred_element_type=jnp.float32) + bfc_ref[...]


def _final(agga, aggb, h2p, dinv, b2r, batch_pad, Wfc, bfcr):
    return pl.pallas_call(
        _final_body,
        out_shape=jax.ShapeDtypeStruct((N_GRAPHS, N_CLS), jnp.float32),
    )(agga, aggb, h2p, dinv, b2r, batch_pad, Wfc, bfcr)


def kernel(x, edge_index, batch, W1, b1, W2, b2, Wfc, bfc):
    src = edge_index[0].astype(jnp.int32)
    dst = edge_index[1].astype(jnp.int32)
    pad = jnp.full((E_PAD - N_EDGES,), PAD_NODE, jnp.int32)
    src_pad = jnp.concatenate([src, pad])
    dst_pad = jnp.concatenate([dst, pad])
    x_pad = jnp.pad(x, ((0, NP - N_NODES), (0, 0)))
    batch_pad = jnp.concatenate(
        [batch.astype(jnp.int32),
         jnp.full((NP - N_NODES,), N_GRAPHS, jnp.int32)]).reshape(1, NP)

    deg_call, agg_call = _sc_kernels()

    deg = deg_call(dst_pad)
    h1p, dinv = _mm1(x_pad, W1, deg[0], deg[1])
    agg1 = agg_call(src_pad, dst_pad, h1p)
    h2p = _mm2(agg1[0], agg1[1], h1p, dinv, b1.reshape(1, HID), W2)
    agg2 = agg_call(src_pad, dst_pad, h2p)
    return _final(agg2[0], agg2[1], h2p, dinv, b2.reshape(1, HID),
                  batch_pad, Wfc, bfc.reshape(1, N_CLS))


# speedup vs baseline: 11.9138x; 11.9138x over previous
import functools

import jax
import jax.numpy as jnp
from jax import lax
from jax.experimental import pallas as pl
from jax.experimental.pallas import tpu as pltpu
from jax.experimental.pallas import tpu_sc as plsc

N_NODES = 10000
N_EDGES = 320000
D_FEAT = 128
HID = 16
N_CLS = 10
N_GRAPHS = 64

NP = 10240
PAD_NODE = N_NODES
N_TILES = 32
CHUNK = 128
EPT = 10112
E_PAD = EPT * N_TILES
NCH = EPT // CHUNK
ROWS_PER_TILE = NP // 16


E_PAD2 = E_PAD // 2
GRID = E_PAD2 // CHUNK
CHUNKR = CHUNK
GRIDR = NP // CHUNKR
DF = 128


@functools.lru_cache(maxsize=None)
def _sc_kernels():
    mesh = plsc.VectorSubcoreMesh(core_axis_name="c", subcore_axis_name="s")

    def _zero_pipe(rows_hbm, zbuf, acc_sh):
        def zbody(ridx):
            pltpu.sync_copy(zbuf.at[pl.ds(0, CHUNKR)], acc_sh.at[ridx.at[0]])

        pltpu.emit_pipeline(
            zbody,
            grid=(GRIDR,),
            in_specs=[pl.BlockSpec((1, CHUNKR), lambda i: (0, i))],
            out_specs=[],
            core_axis_name="s",
            dimension_semantics=(pltpu.PARALLEL,),
        )(rows_hbm)

    STEPS_PER_TILE = GRIDR // 16

    def _drain_pipe(rows_hbm, ridx_v, rows, acc_sh, out_hbm, c, s):
        @pl.loop(0, STEPS_PER_TILE)
        def _(k):
            step = s * STEPS_PER_TILE + k
            pltpu.sync_copy(rows_hbm.at[0].at[pl.ds(step * CHUNKR, CHUNKR)],
                            ridx_v)
            pltpu.sync_copy(acc_sh.at[ridx_v], rows)
            pltpu.sync_copy(
                rows, out_hbm.at[pl.ds(c * NP + step * CHUNKR, CHUNKR)])

    def agg_body(rows_hbm, src2_hbm, dst2_hbm, tab_hbm, out_hbm,
                 rows, ridx_v, acc_sh):
        c = lax.axis_index("c")
        s = lax.axis_index("s")

        @pl.loop(0, CHUNKR)
        def _(i):
            rows[i, :] = jnp.full((DF,), 0.0, jnp.float32)

        _zero_pipe(rows_hbm, rows, acc_sh)
        plsc.subcore_barrier()

        def body(sidx, didx):
            pltpu.sync_copy(tab_hbm.at[sidx.at[0]], rows)
            pltpu.sync_copy(rows, acc_sh.at[didx.at[0]], add=True)

        pltpu.emit_pipeline(
            body,
            grid=(GRID,),
            in_specs=[
                pl.BlockSpec((1, CHUNK), lambda i: (c, i)),
                pl.BlockSpec((1, CHUNK), lambda i: (c, i)),
            ],
            out_specs=[],
            core_axis_name="s",
            dimension_semantics=(pltpu.PARALLEL,),
        )(src2_hbm, dst2_hbm)

        plsc.subcore_barrier()
        _drain_pipe(rows_hbm, ridx_v, rows, acc_sh, out_hbm, c, s)

    agg_call = pl.kernel(
        agg_body,
        out_type=jax.ShapeDtypeStruct((2 * NP, DF), jnp.float32),
        mesh=mesh,
        scratch_types=[
            pltpu.VMEM((CHUNK, DF), jnp.float32),
            pltpu.VMEM((CHUNK,), jnp.int32),
            pltpu.VMEM_SHARED((NP, DF), jnp.float32),
        ],
    )

    def deg_body(rows_hbm, dst2_hbm, out_hbm, ones_v, ridx_v, acc_sh):
        c = lax.axis_index("c")
        s = lax.axis_index("s")

        @pl.loop(0, CHUNKR)
        def _(i):
            ones_v[i, :] = jnp.full((DF,), 0.0, jnp.float32)

        _zero_pipe(rows_hbm, ones_v, acc_sh)

        @pl.loop(0, CHUNK)
        def _(i):
            ones_v[i, :] = jnp.full((DF,), 1.0, jnp.float32)

        plsc.subcore_barrier()

        def body(didx):
            pltpu.sync_copy(ones_v, acc_sh.at[didx.at[0]], add=True)

        pltpu.emit_pipeline(
            body,
            grid=(GRID,),
            in_specs=[pl.BlockSpec((1, CHUNK), lambda i: (c, i))],
            out_specs=[],
            core_axis_name="s",
            dimension_semantics=(pltpu.PARALLEL,),
        )(dst2_hbm)

        plsc.subcore_barrier()
        _drain_pipe(rows_hbm, ridx_v, ones_v, acc_sh, out_hbm, c, s)

    deg_call = pl.kernel(
        deg_body,
        out_type=jax.ShapeDtypeStruct((2 * NP, DF), jnp.float32),
        mesh=mesh,
        scratch_types=[
            pltpu.VMEM((CHUNK, DF), jnp.float32),
            pltpu.VMEM((CHUNK,), jnp.int32),
            pltpu.VMEM_SHARED((NP, DF), jnp.float32),
        ],
    )

    return deg_call, agg_call


def _mm1_body(x_ref, dega_ref, degb_ref, xp_ref, dinv_ref):
    deg = dega_ref[...] + degb_ref[...] + 1.0
    dinv128 = lax.rsqrt(deg)
    xp_ref[...] = x_ref[...] * dinv128
    dinv_ref[...] = dinv128[:, :HID]


def _mm1(x_pad, dega, degb):
    BM = 2048
    return pl.pallas_call(
        _mm1_body,
        grid=(NP // BM,),
        in_specs=[
            pl.BlockSpec((BM, DF), lambda i: (i, 0)),
            pl.BlockSpec((BM, DF), lambda i: (i, 0)),
            pl.BlockSpec((BM, DF), lambda i: (i, 0)),
        ],
        out_specs=[
            pl.BlockSpec((BM, DF), lambda i: (i, 0)),
            pl.BlockSpec((BM, HID), lambda i: (i, 0)),
        ],
        out_shape=[
            jax.ShapeDtypeStruct((NP, DF), jnp.float32),
            jax.ShapeDtypeStruct((NP, HID), jnp.float32),
        ],
    )(x_pad, dega, degb)


def _mm2_body(agga_ref, aggb_ref, xp_ref, dinv_ref, w1_ref, b1_ref, w2_ref,
              t2_ref):
    a1 = agga_ref[...] + aggb_ref[...] + xp_ref[...]
    dinv = dinv_ref[...]
    h1 = jnp.dot(a1, w1_ref[...], preferred_element_type=jnp.float32)
    r = jnp.maximum(dinv * h1 + b1_ref[...], 0.0)
    h2 = jnp.dot(r, w2_ref[...], preferred_element_type=jnp.float32)
    h2p = h2 * dinv
    t2_ref[...] = jnp.pad(h2p, ((0, 0), (0, DF - HID)))


def _mm2(agga, aggb, xp, dinv, W1, b1r, W2):
    BM = 2048
    return pl.pallas_call(
        _mm2_body,
        grid=(NP // BM,),
        in_specs=[
            pl.BlockSpec((BM, DF), lambda i: (i, 0)),
            pl.BlockSpec((BM, DF), lambda i: (i, 0)),
            pl.BlockSpec((BM, DF), lambda i: (i, 0)),
            pl.BlockSpec((BM, HID), lambda i: (i, 0)),
            pl.BlockSpec((DF, HID), lambda i: (0, 0)),
            pl.BlockSpec((1, HID), lambda i: (0, 0)),
            pl.BlockSpec((HID, HID), lambda i: (0, 0)),
        ],
        out_specs=pl.BlockSpec((BM, DF), lambda i: (i, 0)),
        out_shape=jax.ShapeDtypeStruct((NP, DF), jnp.float32),
    )(agga, aggb, xp, dinv, W1, b1r, W2)


def _final_body(agga_ref, aggb_ref, t2_ref, dinv_ref, b2_ref, batch_ref,
                wfc_ref, bfc_ref, out_ref):
    out2 = dinv_ref[...] * (agga_ref[..., :HID] + aggb_ref[..., :HID]
                            + t2_ref[..., :HID]) + b2_ref[...]
    ids = lax.broadcasted_iota(jnp.int32, (N_GRAPHS, NP), 0)
    sel = (ids == batch_ref[...]).astype(jnp.float32)
    sums = jnp.dot(sel, out2, preferred_element_type=jnp.float32)
    counts = jnp.sum(sel, axis=1, keepdims=True)
    mean = sums / jnp.maximum(counts, 1.0)
    out_ref[...] = jnp.dot(mean, wfc_ref[...],
                           preferred_element_type=jnp.float32) + bfc_ref[...]


def _final(agga, aggb, t2, dinv, b2r, batch_pad, Wfc, bfcr):
    return pl.pallas_call(
        _final_body,
        out_shape=jax.ShapeDtypeStruct((N_GRAPHS, N_CLS), jnp.float32),
    )(agga, aggb, t2, dinv, b2r, batch_pad, Wfc, bfcr)


def kernel(x, edge_index, batch, W1, b1, W2, b2, Wfc, bfc):
    src = edge_index[0].astype(jnp.int32)
    dst = edge_index[1].astype(jnp.int32)
    pad = jnp.full((E_PAD - N_EDGES,), PAD_NODE, jnp.int32)
    src2 = jnp.concatenate([src, pad]).reshape(2, E_PAD2)
    dst2 = jnp.concatenate([dst, pad]).reshape(2, E_PAD2)
    x_pad = jnp.pad(x, ((0, NP - N_NODES), (0, 0)))
    batch_pad = jnp.concatenate(
        [batch.astype(jnp.int32),
         jnp.full((NP - N_NODES,), N_GRAPHS, jnp.int32)]).reshape(1, NP)
    row_ids = jnp.arange(NP, dtype=jnp.int32).reshape(1, NP)

    deg_call, agg_call = _sc_kernels()

    deg = deg_call(row_ids, dst2).reshape(2, NP, DF)
    xp, dinv = _mm1(x_pad, deg[0], deg[1])
    agg1 = agg_call(row_ids, src2, dst2, xp).reshape(2, NP, DF)
    t2 = _mm2(agg1[0], agg1[1], xp, dinv, W1, b1.reshape(1, HID), W2)
    agg2 = agg_call(row_ids, src2, dst2, t2).reshape(2, NP, DF)
    return _final(agg2[0], agg2[1], t2, dinv, b2.reshape(1, HID),
                  batch_pad, Wfc, bfc.reshape(1, N_CLS))


# speedup vs baseline: 12.5184x; 1.0508x over previous
import functools

import jax
import jax.numpy as jnp
from jax import lax
from jax.experimental import pallas as pl
from jax.experimental.pallas import tpu as pltpu
from jax.experimental.pallas import tpu_sc as plsc

N_NODES = 10000
N_EDGES = 320000
D_FEAT = 128
HID = 16
N_CLS = 10
N_GRAPHS = 64

NP = 10240
PAD_NODE = N_NODES
N_TILES = 32
CHUNK = 128
EPT = 10112
E_PAD = EPT * N_TILES
NCH = EPT // CHUNK
ROWS_PER_TILE = NP // 16


E_PAD2 = E_PAD // 2
GRID = E_PAD2 // CHUNK
CHUNKR = CHUNK
GRIDR = NP // CHUNKR
DF = 128


@functools.lru_cache(maxsize=None)
def _sc_kernels():
    mesh = plsc.VectorSubcoreMesh(core_axis_name="c", subcore_axis_name="s")

    def _zero_pipe(rows_hbm, zbuf, acc_sh):
        def zbody(ridx):
            pltpu.sync_copy(zbuf.at[pl.ds(0, CHUNKR)], acc_sh.at[ridx.at[0]])

        pltpu.emit_pipeline(
            zbody,
            grid=(GRIDR,),
            in_specs=[pl.BlockSpec((1, CHUNKR), lambda i: (0, i))],
            out_specs=[],
            core_axis_name="s",
            dimension_semantics=(pltpu.PARALLEL,),
        )(rows_hbm)

    STEPS_PER_TILE = GRIDR // 16

    def _drain_pipe(rows_hbm, ridx_v, rows, acc_sh, out_hbm, c, s):
        @pl.loop(0, STEPS_PER_TILE)
        def _(k):
            step = s * STEPS_PER_TILE + k
            pltpu.sync_copy(rows_hbm.at[0].at[pl.ds(step * CHUNKR, CHUNKR)],
                            ridx_v)
            pltpu.sync_copy(acc_sh.at[ridx_v], rows)
            pltpu.sync_copy(
                rows, out_hbm.at[pl.ds(c * NP + step * CHUNKR, CHUNKR)])

    def agg_body(rows_hbm, src2_hbm, dst2_hbm, tab_hbm, out_hbm,
                 rows, rows2, ridx_v, acc_sh, gsem0, gsem1):
        c = lax.axis_index("c")
        s = lax.axis_index("s")

        @pl.loop(0, CHUNKR)
        def _(i):
            rows[i, :] = jnp.full((DF,), 0.0, jnp.float32)

        _zero_pipe(rows_hbm, rows, acc_sh)
        plsc.subcore_barrier()

        def body(sidx0, didx0, sidx1, didx1):
            g0 = pltpu.async_copy(tab_hbm.at[sidx0.at[0]], rows, gsem0)
            g1 = pltpu.async_copy(tab_hbm.at[sidx1.at[0]], rows2, gsem1)
            g0.wait()
            pltpu.sync_copy(rows, acc_sh.at[didx0.at[0]], add=True)
            g1.wait()
            pltpu.sync_copy(rows2, acc_sh.at[didx1.at[0]], add=True)

        pltpu.emit_pipeline(
            body,
            grid=(GRID // 2,),
            in_specs=[
                pl.BlockSpec((1, CHUNK), lambda i: (c, 2 * i)),
                pl.BlockSpec((1, CHUNK), lambda i: (c, 2 * i)),
                pl.BlockSpec((1, CHUNK), lambda i: (c, 2 * i + 1)),
                pl.BlockSpec((1, CHUNK), lambda i: (c, 2 * i + 1)),
            ],
            out_specs=[],
            core_axis_name="s",
            dimension_semantics=(pltpu.PARALLEL,),
        )(src2_hbm, dst2_hbm, src2_hbm, dst2_hbm)

        plsc.subcore_barrier()
        _drain_pipe(rows_hbm, ridx_v, rows, acc_sh, out_hbm, c, s)

    agg_call = pl.kernel(
        agg_body,
        out_type=jax.ShapeDtypeStruct((2 * NP, DF), jnp.float32),
        mesh=mesh,
        scratch_types=[
            pltpu.VMEM((CHUNK, DF), jnp.float32),
            pltpu.VMEM((CHUNK, DF), jnp.float32),
            pltpu.VMEM((CHUNK,), jnp.int32),
            pltpu.VMEM_SHARED((NP, DF), jnp.float32),
            pltpu.SemaphoreType.DMA,
            pltpu.SemaphoreType.DMA,
        ],
    )

    def deg_body(rows_hbm, dst2_hbm, out_hbm, ones_v, ridx_v, acc_sh):
        c = lax.axis_index("c")
        s = lax.axis_index("s")

        @pl.loop(0, CHUNKR)
        def _(i):
            ones_v[i, :] = jnp.full((DF,), 0.0, jnp.float32)

        _zero_pipe(rows_hbm, ones_v, acc_sh)

        @pl.loop(0, CHUNK)
        def _(i):
            ones_v[i, :] = jnp.full((DF,), 1.0, jnp.float32)

        plsc.subcore_barrier()

        def body(didx):
            pltpu.sync_copy(ones_v, acc_sh.at[didx.at[0]], add=True)

        pltpu.emit_pipeline(
            body,
            grid=(GRID,),
            in_specs=[pl.BlockSpec((1, CHUNK), lambda i: (c, i))],
            out_specs=[],
            core_axis_name="s",
            dimension_semantics=(pltpu.PARALLEL,),
        )(dst2_hbm)

        plsc.subcore_barrier()
        _drain_pipe(rows_hbm, ridx_v, ones_v, acc_sh, out_hbm, c, s)

    deg_call = pl.kernel(
        deg_body,
        out_type=jax.ShapeDtypeStruct((2 * NP, DF), jnp.float32),
        mesh=mesh,
        scratch_types=[
            pltpu.VMEM((CHUNK, DF), jnp.float32),
            pltpu.VMEM((CHUNK,), jnp.int32),
            pltpu.VMEM_SHARED((NP, DF), jnp.float32),
        ],
    )

    return deg_call, agg_call


def _mm1_body(x_ref, dega_ref, degb_ref, xp_ref, dinv_ref):
    deg = dega_ref[...] + degb_ref[...] + 1.0
    dinv128 = lax.rsqrt(deg)
    xp_ref[...] = x_ref[...] * dinv128
    dinv_ref[...] = dinv128[:, :HID]


def _mm1(x_pad, dega, degb):
    BM = 2048
    return pl.pallas_call(
        _mm1_body,
        grid=(NP // BM,),
        in_specs=[
            pl.BlockSpec((BM, DF), lambda i: (i, 0)),
            pl.BlockSpec((BM, DF), lambda i: (i, 0)),
            pl.BlockSpec((BM, DF), lambda i: (i, 0)),
        ],
        out_specs=[
            pl.BlockSpec((BM, DF), lambda i: (i, 0)),
            pl.BlockSpec((BM, HID), lambda i: (i, 0)),
        ],
        out_shape=[
            jax.ShapeDtypeStruct((NP, DF), jnp.float32),
            jax.ShapeDtypeStruct((NP, HID), jnp.float32),
        ],
    )(x_pad, dega, degb)


def _mm2_body(agga_ref, aggb_ref, xp_ref, dinv_ref, w1_ref, b1_ref, w2_ref,
              t2_ref):
    a1 = agga_ref[...] + aggb_ref[...] + xp_ref[...]
    dinv = dinv_ref[...]
    h1 = jnp.dot(a1, w1_ref[...], preferred_element_type=jnp.float32)
    r = jnp.maximum(dinv * h1 + b1_ref[...], 0.0)
    h2 = jnp.dot(r, w2_ref[...], preferred_element_type=jnp.float32)
    h2p = h2 * dinv
    t2_ref[...] = jnp.pad(h2p, ((0, 0), (0, DF - HID)))


def _mm2(agga, aggb, xp, dinv, W1, b1r, W2):
    BM = 2048
    return pl.pallas_call(
        _mm2_body,
        grid=(NP // BM,),
        in_specs=[
            pl.BlockSpec((BM, DF), lambda i: (i, 0)),
            pl.BlockSpec((BM, DF), lambda i: (i, 0)),
            pl.BlockSpec((BM, DF), lambda i: (i, 0)),
            pl.BlockSpec((BM, HID), lambda i: (i, 0)),
            pl.BlockSpec((DF, HID), lambda i: (0, 0)),
            pl.BlockSpec((1, HID), lambda i: (0, 0)),
            pl.BlockSpec((HID, HID), lambda i: (0, 0)),
        ],
        out_specs=pl.BlockSpec((BM, DF), lambda i: (i, 0)),
        out_shape=jax.ShapeDtypeStruct((NP, DF), jnp.float32),
    )(agga, aggb, xp, dinv, W1, b1r, W2)


def _final_body(agga_ref, aggb_ref, t2_ref, dinv_ref, b2_ref, batch_ref,
                wfc_ref, bfc_ref, out_ref):
    out2 = dinv_ref[...] * (agga_ref[..., :HID] + aggb_ref[..., :HID]
                            + t2_ref[..., :HID]) + b2_ref[...]
    ids = lax.broadcasted_iota(jnp.int32, (N_GRAPHS, NP), 0)
    sel = (ids == batch_ref[...]).astype(jnp.float32)
    sums = jnp.dot(sel, out2, preferred_element_type=jnp.float32)
    counts = jnp.sum(sel, axis=1, keepdims=True)
    mean = sums / jnp.maximum(counts, 1.0)
    out_ref[...] = jnp.dot(mean, wfc_ref[...],
                           preferred_element_type=jnp.float32) + bfc_ref[...]


def _final(agga, aggb, t2, dinv, b2r, batch_pad, Wfc, bfcr):
    return pl.pallas_call(
        _final_body,
        out_shape=jax.ShapeDtypeStruct((N_GRAPHS, N_CLS), jnp.float32),
    )(agga, aggb, t2, dinv, b2r, batch_pad, Wfc, bfcr)


def kernel(x, edge_index, batch, W1, b1, W2, b2, Wfc, bfc):
    src = edge_index[0].astype(jnp.int32)
    dst = edge_index[1].astype(jnp.int32)
    pad = jnp.full((E_PAD - N_EDGES,), PAD_NODE, jnp.int32)
    src2 = jnp.concatenate([src, pad]).reshape(2, E_PAD2)
    dst2 = jnp.concatenate([dst, pad]).reshape(2, E_PAD2)
    x_pad = jnp.pad(x, ((0, NP - N_NODES), (0, 0)))
    batch_pad = jnp.concatenate(
        [batch.astype(jnp.int32),
         jnp.full((NP - N_NODES,), N_GRAPHS, jnp.int32)]).reshape(1, NP)
    row_ids = jnp.arange(NP, dtype=jnp.int32).reshape(1, NP)

    deg_call, agg_call = _sc_kernels()

    deg = deg_call(row_ids, dst2).reshape(2, NP, DF)
    xp, dinv = _mm1(x_pad, deg[0], deg[1])
    agg1 = agg_call(row_ids, src2, dst2, xp).reshape(2, NP, DF)
    t2 = _mm2(agg1[0], agg1[1], xp, dinv, W1, b1.reshape(1, HID), W2)
    agg2 = agg_call(row_ids, src2, dst2, t2).reshape(2, NP, DF)
    return _final(agg2[0], agg2[1], t2, dinv, b2.reshape(1, HID),
                  batch_pad, Wfc, bfc.reshape(1, N_CLS))
